# baseline (speedup 1.0000x reference)
import functools

import jax
import jax.numpy as jnp
from jax import lax
from jax.experimental import pallas as pl
from jax.experimental.pallas import tpu as pltpu

_BN_EPS = 1e-5
_CHUNK = 512
_DEPTH_IN = 4
_DEPTH_OUT = 2


def _pick_l_tile(length, cap=2048):
    if length % 128 == 0:
        t = min(cap, length)
        t -= t % 128
        while length % t:
            t -= 128
        return t
    return length


def _fused_kernel(w_ref, vec_ref, x_hbm, o_hbm,
                  ring_ref, in_sem, ostage_ref, out_sem,
                  stash_ref, sum_ref, gram_ref, wf_ref, bf_ref,
                  *, m, n_l, blk, l_tile, nbl):
    p = pl.program_id(0)
    b = pl.program_id(1)
    l = pl.program_id(2)
    bl = b * n_l + l
    d_in = _DEPTH_IN if nbl >= _DEPTH_IN else 1
    d_out = _DEPTH_OUT if nbl >= _DEPTH_OUT else 1

    def split_idx(idx):
        if n_l == 1:
            return idx, 0
        return idx // n_l, idx % n_l

    def in_copy(idx, slot):
        bb, ll = split_idx(idx)
        return pltpu.make_async_copy(
            x_hbm.at[pl.ds(bb * blk, blk), :, pl.ds(ll * l_tile, l_tile)],
            ring_ref.at[slot],
            in_sem.at[slot],
        )

    def out_copy(idx, slot):
        bb, ll = split_idx(idx)
        return pltpu.make_async_copy(
            ostage_ref.at[slot],
            o_hbm.at[pl.ds(bb * blk, blk), :, pl.ds(ll * l_tile, l_tile)],
            out_sem.at[slot],
        )

    @pl.when((p == 0) & (bl == 0))
    def _init():
        sum_ref[...] = jnp.zeros_like(sum_ref)
        gram_ref[...] = jnp.zeros_like(gram_ref)
        for d in range(d_in):
            in_copy(d, d).start()

    @pl.when(p == 0)
    def _stats():
        slot = jnp.bitwise_and(bl, d_in - 1)
        in_copy(0, slot).wait()
        ch = min(_CHUNK, l_tile)
        acc_g = jnp.zeros_like(gram_ref)
        acc_v = jnp.zeros((ring_ref.shape[2], ch), jnp.float32)
        for i in range(blk):
            for c in range(l_tile // ch):
                x = ring_ref[slot, i, :, c * ch:(c + 1) * ch]
                xb = x.astype(jnp.bfloat16)
                stash_ref[bl * blk + i, :, c * ch:(c + 1) * ch] = xb
                acc_v += x
                acc_g += lax.dot_general(
                    xb, xb, (((1,), (1,)), ((), ())),
                    preferred_element_type=jnp.float32)
        sum_ref[...] += jnp.sum(acc_v, axis=1, keepdims=True)
        gram_ref[...] += acc_g
        @pl.when(bl + d_in < nbl)
        def _():
            in_copy(bl + d_in, slot).start()

    @pl.when((p == 1) & (bl == 0))
    def _fold():
        sum_x = sum_ref[...]
        gram = gram_ref[...]
        inv_m = 1.0 / m
        mean_x = sum_x * inv_m
        outer = lax.dot_general(
            mean_x, mean_x, (((1,), (1,)), ((), ())),
            preferred_element_type=jnp.float32)
        cov = gram * inv_m - outer
        w = w_ref[...]
        bias = vec_ref[:, 0:1]
        gamma = vec_ref[:, 1:2]
        beta = vec_ref[:, 2:3]
        mean_y = jnp.dot(w, mean_x, preferred_element_type=jnp.float32)
        mean_y = mean_y + bias
        v = jnp.dot(w, cov, preferred_element_type=jnp.float32)
        var_y = jnp.sum(v * w, axis=1, keepdims=True)
        scale = gamma * lax.rsqrt(var_y + _BN_EPS)
        wf_ref[...] = (w * scale).astype(jnp.bfloat16)
        bf_ref[...] = beta + scale * (bias - mean_y)

    @pl.when(p == 1)
    def _matmul():
        wf = wf_ref[...]
        bf = bf_ref[...]
        ch = min(1024, l_tile)
        slot = jnp.bitwise_and(bl, d_out - 1)
        @pl.when(bl >= d_out)
        def _():
            out_copy(0, slot).wait()
        for c in range(l_tile // ch):
            for i in range(blk):
                xb = stash_ref[bl * blk + i, :, c * ch:(c + 1) * ch]
                y = jnp.dot(wf, xb, preferred_element_type=jnp.float32)
                ostage_ref[slot, i, :, c * ch:(c + 1) * ch] = jnp.maximum(
                    y + bf, 0.0).astype(ostage_ref.dtype)
        out_copy(bl, slot).start()
        @pl.when(bl == nbl - 1)
        def _():
            for d in range(d_out):
                out_copy(0, d).wait()


def kernel(x, weight, bias, gamma, beta):
    n, c_in, length = x.shape
    c_out = weight.shape[0]
    m = n * length

    l_tile = _pick_l_tile(length)
    n_l = length // l_tile

    blk = 2 if n % 2 == 0 else 1
    nb = n // blk
    nbl = nb * n_l

    w2 = weight[:, :, 0].astype(jnp.float32)
    vecs = jnp.stack([bias, gamma, beta], axis=1).astype(jnp.float32)

    d_in = _DEPTH_IN if nbl >= _DEPTH_IN else 1
    d_out = _DEPTH_OUT if nbl >= _DEPTH_OUT else 1

    out = pl.pallas_call(
        functools.partial(_fused_kernel, m=float(m), n_l=n_l, blk=blk,
                          l_tile=l_tile, nbl=nbl),
        out_shape=jax.ShapeDtypeStruct((n, c_out, length), x.dtype),
        grid_spec=pltpu.PrefetchScalarGridSpec(
            num_scalar_prefetch=0,
            grid=(2, nb, n_l),
            in_specs=[
                pl.BlockSpec((c_out, c_in), lambda p, b, l: (0, 0)),
                pl.BlockSpec((c_out, 3), lambda p, b, l: (0, 0)),
                pl.BlockSpec(memory_space=pl.ANY),
            ],
            out_specs=pl.BlockSpec(memory_space=pl.ANY),
            scratch_shapes=[
                pltpu.VMEM((d_in, blk, c_in, l_tile), jnp.float32),
                pltpu.SemaphoreType.DMA((d_in,)),
                pltpu.VMEM((d_out, blk, c_out, l_tile), jnp.float32),
                pltpu.SemaphoreType.DMA((d_out,)),
                pltpu.VMEM((n * n_l, c_in, l_tile), jnp.bfloat16),
                pltpu.VMEM((c_in, 1), jnp.float32),
                pltpu.VMEM((c_in, c_in), jnp.float32),
                pltpu.VMEM((c_out, c_in), jnp.bfloat16),
                pltpu.VMEM((c_out, 1), jnp.float32),
            ],
        ),
        compiler_params=pltpu.CompilerParams(
            dimension_semantics=("arbitrary", "arbitrary", "arbitrary")),
    )(w2, vecs, x)

    return out


# speedup vs baseline: 1.0087x; 1.0087x over previous
import functools

import jax
import jax.numpy as jnp
from jax import lax
from jax.experimental import pallas as pl
from jax.experimental.pallas import tpu as pltpu

_BN_EPS = 1e-5
_CHUNK = 512
_DEPTH_IN = 3
_DEPTH_OUT = 2


def _slot(idx, depth):
    if depth & (depth - 1) == 0:
        return jnp.bitwise_and(idx, depth - 1)
    return lax.rem(idx, depth)


def _pick_l_tile(length, cap=2048):
    if length % 128 == 0:
        t = min(cap, length)
        t -= t % 128
        while length % t:
            t -= 128
        return t
    return length


def _fused_kernel(w_ref, vec_ref, x_hbm, o_hbm,
                  ring_ref, in_sem, ostage_ref, out_sem,
                  stash_ref, sum_ref, gram_ref, wf_ref, bf_ref,
                  *, m, n_l, blk, l_tile, nbl):
    p = pl.program_id(0)
    b = pl.program_id(1)
    l = pl.program_id(2)
    bl = b * n_l + l
    d_in = _DEPTH_IN if nbl >= _DEPTH_IN else 1
    d_out = _DEPTH_OUT if nbl >= _DEPTH_OUT else 1

    def split_idx(idx):
        if n_l == 1:
            return idx, 0
        return idx // n_l, idx % n_l

    def in_copy(idx, slot):
        bb, ll = split_idx(idx)
        return pltpu.make_async_copy(
            x_hbm.at[pl.ds(bb * blk, blk), :, pl.ds(ll * l_tile, l_tile)],
            ring_ref.at[slot],
            in_sem.at[slot],
        )

    def out_copy(idx, slot):
        bb, ll = split_idx(idx)
        return pltpu.make_async_copy(
            ostage_ref.at[slot],
            o_hbm.at[pl.ds(bb * blk, blk), :, pl.ds(ll * l_tile, l_tile)],
            out_sem.at[slot],
        )

    @pl.when((p == 0) & (bl == 0))
    def _init():
        sum_ref[...] = jnp.zeros_like(sum_ref)
        gram_ref[...] = jnp.zeros_like(gram_ref)
        for d in range(d_in):
            in_copy(d, d).start()

    @pl.when(p == 0)
    def _stats():
        slot = _slot(bl, d_in)
        in_copy(0, slot).wait()
        ch = min(_CHUNK, l_tile)
        acc_g = jnp.zeros_like(gram_ref)
        acc_v = jnp.zeros((ring_ref.shape[2], ch), jnp.float32)
        for i in range(blk):
            for c in range(l_tile // ch):
                x = ring_ref[slot, i, :, c * ch:(c + 1) * ch]
                xb = x.astype(jnp.bfloat16)
                stash_ref[bl * blk + i, :, c * ch:(c + 1) * ch] = xb
                acc_v += x
                acc_g += lax.dot_general(
                    xb, xb, (((1,), (1,)), ((), ())),
                    preferred_element_type=jnp.float32)
        sum_ref[...] += jnp.sum(acc_v, axis=1, keepdims=True)
        gram_ref[...] += acc_g
        @pl.when(bl + d_in < nbl)
        def _():
            in_copy(bl + d_in, slot).start()

    @pl.when((p == 1) & (bl == 0))
    def _fold():
        sum_x = sum_ref[...]
        gram = gram_ref[...]
        inv_m = 1.0 / m
        mean_x = sum_x * inv_m
        outer = lax.dot_general(
            mean_x, mean_x, (((1,), (1,)), ((), ())),
            preferred_element_type=jnp.float32)
        cov = gram * inv_m - outer
        w = w_ref[...]
        bias = vec_ref[:, 0:1]
        gamma = vec_ref[:, 1:2]
        beta = vec_ref[:, 2:3]
        mean_y = jnp.dot(w, mean_x, preferred_element_type=jnp.float32)
        mean_y = mean_y + bias
        v = jnp.dot(w, cov, preferred_element_type=jnp.float32)
        var_y = jnp.sum(v * w, axis=1, keepdims=True)
        scale = gamma * lax.rsqrt(var_y + _BN_EPS)
        wf_ref[...] = (w * scale).astype(jnp.bfloat16)
        bf_ref[...] = beta + scale * (bias - mean_y)

    @pl.when(p == 1)
    def _matmul():
        wf = wf_ref[...]
        bf = bf_ref[...]
        ch = min(1024, l_tile)
        slot = _slot(bl, d_out)
        @pl.when(bl >= d_out)
        def _():
            out_copy(0, slot).wait()
        for c in range(l_tile // ch):
            for i in range(blk):
                xb = stash_ref[bl * blk + i, :, c * ch:(c + 1) * ch]
                y = jnp.dot(wf, xb, preferred_element_type=jnp.float32)
                ostage_ref[slot, i, :, c * ch:(c + 1) * ch] = jnp.maximum(
                    y + bf, 0.0).astype(ostage_ref.dtype)
        out_copy(bl, slot).start()
        @pl.when(bl == nbl - 1)
        def _():
            for d in range(d_out):
                out_copy(0, d).wait()


def kernel(x, weight, bias, gamma, beta):
    n, c_in, length = x.shape
    c_out = weight.shape[0]
    m = n * length

    l_tile = _pick_l_tile(length)
    n_l = length // l_tile

    blk = 2 if n % 2 == 0 else 1
    nb = n // blk
    nbl = nb * n_l

    w2 = weight[:, :, 0].astype(jnp.float32)
    vecs = jnp.stack([bias, gamma, beta], axis=1).astype(jnp.float32)

    d_in = _DEPTH_IN if nbl >= _DEPTH_IN else 1
    d_out = _DEPTH_OUT if nbl >= _DEPTH_OUT else 1

    out = pl.pallas_call(
        functools.partial(_fused_kernel, m=float(m), n_l=n_l, blk=blk,
                          l_tile=l_tile, nbl=nbl),
        out_shape=jax.ShapeDtypeStruct((n, c_out, length), x.dtype),
        grid_spec=pltpu.PrefetchScalarGridSpec(
            num_scalar_prefetch=0,
            grid=(2, nb, n_l),
            in_specs=[
                pl.BlockSpec((c_out, c_in), lambda p, b, l: (0, 0)),
                pl.BlockSpec((c_out, 3), lambda p, b, l: (0, 0)),
                pl.BlockSpec(memory_space=pl.ANY),
            ],
            out_specs=pl.BlockSpec(memory_space=pl.ANY),
            scratch_shapes=[
                pltpu.VMEM((d_in, blk, c_in, l_tile), jnp.float32),
                pltpu.SemaphoreType.DMA((d_in,)),
                pltpu.VMEM((d_out, blk, c_out, l_tile), jnp.float32),
                pltpu.SemaphoreType.DMA((d_out,)),
                pltpu.VMEM((n * n_l, c_in, l_tile), jnp.bfloat16),
                pltpu.VMEM((c_in, 1), jnp.float32),
                pltpu.VMEM((c_in, c_in), jnp.float32),
                pltpu.VMEM((c_out, c_in), jnp.bfloat16),
                pltpu.VMEM((c_out, 1), jnp.float32),
            ],
        ),
        compiler_params=pltpu.CompilerParams(
            dimension_semantics=("arbitrary", "arbitrary", "arbitrary")),
    )(w2, vecs, x)

    return out
